# baseline (speedup 1.0000x reference)
import functools

import jax
import jax.numpy as jnp
from jax import lax
from jax.experimental import pallas as pl
from jax.experimental.pallas import tpu as pltpu
from jax.experimental.pallas import tpu_sc as plsc

F32 = jnp.float32
_INV_S = 1.0 / (1.0 + 1e-5) ** 0.5
_MAXN = 1.0 - 1e-5
_PREC = lax.Precision.HIGHEST


def _rnorm(x):
    return jnp.maximum(jnp.sqrt(jnp.sum(x * x, axis=-1, keepdims=True)), 1e-15)


def _artanh(x):
    x = jnp.clip(x, -1.0 + 1e-7, 1.0 - 1e-7)
    return 0.5 * (jnp.log1p(x) - jnp.log1p(-x))


def _proj(x):
    n = _rnorm(x)
    return jnp.where(n > _MAXN, x / n * _MAXN, x)


def _expmap0(u):
    n = _rnorm(u)
    return jnp.tanh(n) * u / n


def _logmap0(p):
    n = _rnorm(p)
    return _artanh(n) * p / n


def _mobius_add(x, y):
    x2 = jnp.sum(x * x, -1, keepdims=True)
    y2 = jnp.sum(y * y, -1, keepdims=True)
    xy = jnp.sum(x * y, -1, keepdims=True)
    num = (1 + 2 * xy + y2) * x + (1 - x2) * y
    den = 1 + 2 * xy + x2 * y2
    return num / jnp.maximum(den, 1e-15)


def _matvecT(x, W):
    return lax.dot_general(x, W, (((1,), (1,)), ((), ())),
                           precision=_PREC, preferred_element_type=F32)


def _mobius_matvec_block(y, W):
    xn = _rnorm(y)
    mx = _matvecT(y, W)
    mn = _rnorm(mx)
    return jnp.tanh(mn / xn * _artanh(xn)) * mx / mn


def _stage_a_body(x_ref, w1_ref, b1_ref, w2_ref, b2_ref, gw_ref, gb_ref,
                  out_ref):
    x = x_ref[0]
    h = jax.nn.relu((_matvecT(x, w1_ref[0]) + b1_ref[0]) * _INV_S)
    h = jax.nn.relu((_matvecT(h, w2_ref[0]) + b2_ref[0]) * _INV_S)
    y = _proj(_expmap0(h))
    mv = _proj(_mobius_matvec_block(y, gw_ref[...]))
    gb = _proj(_expmap0(gb_ref[...]))
    h2 = _proj(_mobius_add(mv, gb))
    out_ref[...] = _logmap0(h2)


def _run_stage_a(Xs, TW1, Tb1, TW2, Tb2, gW, gb, NP):
    T, P, d = Xs.shape
    PB = P // 128
    grid = (T, PB)
    return pl.pallas_call(
        _stage_a_body,
        grid=grid,
        in_specs=[
            pl.BlockSpec((1, 128, d), lambda t, i: (t, i, 0)),
            pl.BlockSpec((1, d, d), lambda t, i: (t, 0, 0)),
            pl.BlockSpec((1, 1, d), lambda t, i: (t, 0, 0)),
            pl.BlockSpec((1, d, d), lambda t, i: (t, 0, 0)),
            pl.BlockSpec((1, 1, d), lambda t, i: (t, 0, 0)),
            pl.BlockSpec((d, d), lambda t, i: (0, 0)),
            pl.BlockSpec((1, d), lambda t, i: (0, 0)),
        ],
        out_specs=pl.BlockSpec((128, d), lambda t, i: (t * PB + i, 0)),
        out_shape=jax.ShapeDtypeStruct((NP, d), F32),
    )(Xs, TW1, Tb1, TW2, Tb2, gW, gb)


def _stage_d_body(p0_ref, p1_ref, xt_ref, gw_ref, gb_ref, out_ref):
    agg = p0_ref[0] + p1_ref[0] + xt_ref[...]
    h = _proj(_expmap0(agg))
    h = _proj(_expmap0(jax.nn.relu(_logmap0(h))))
    mv = _proj(_mobius_matvec_block(h, gw_ref[...]))
    gb = _proj(_expmap0(gb_ref[...]))
    out_ref[...] = _logmap0(_proj(_mobius_add(mv, gb)))


def _run_stage_d(parts, xt1, gW, gb):
    NP, d = xt1.shape
    grid = (NP // 128,)
    return pl.pallas_call(
        _stage_d_body,
        grid=grid,
        in_specs=[
            pl.BlockSpec((1, 128, d), lambda i: (0, i, 0)),
            pl.BlockSpec((1, 128, d), lambda i: (1, i, 0)),
            pl.BlockSpec((128, d), lambda i: (i, 0)),
            pl.BlockSpec((d, d), lambda i: (0, 0)),
            pl.BlockSpec((1, d), lambda i: (0, 0)),
        ],
        out_specs=pl.BlockSpec((128, d), lambda i: (i, 0)),
        out_shape=jax.ShapeDtypeStruct((NP, d), F32),
    )(parts, parts, xt1, gW, gb)


def _stage_f_body(p0_ref, p1_ref, xt_ref, sel_ref, o1_ref, o2_ref, acc_ref):
    i = pl.program_id(0)
    SEL = acc_ref.shape[0]
    B = o1_ref.shape[0]

    @pl.when(i == 0)
    def _():
        acc_ref[...] = jnp.zeros_like(acc_ref)

    blk = p0_ref[0] + p1_ref[0] + xt_ref[...]
    col = lax.broadcasted_iota(jnp.int32, (SEL, 128), 1) + i * 128
    oh = (sel_ref[...] == col).astype(F32)
    acc_ref[...] += lax.dot_general(oh, blk, (((1,), (0,)), ((), ())),
                                    precision=_PREC,
                                    preferred_element_type=F32)

    @pl.when(i == pl.num_programs(0) - 1)
    def _():
        agg = acc_ref[...]
        h = _proj(_expmap0(agg))
        h = _proj(_expmap0(jax.nn.relu(_logmap0(h))))
        res = _proj(_expmap0(_logmap0(h)))
        o1_ref[...] = res[:B]
        o2_ref[...] = res[B:2 * B]


def _run_stage_f(parts, xt2, sel_b, B):
    NP, d = xt2.shape
    SEL = sel_b.shape[0]
    grid = (NP // 128,)
    return pl.pallas_call(
        _stage_f_body,
        grid=grid,
        in_specs=[
            pl.BlockSpec((1, 128, d), lambda i: (0, i, 0)),
            pl.BlockSpec((1, 128, d), lambda i: (1, i, 0)),
            pl.BlockSpec((128, d), lambda i: (i, 0)),
            pl.BlockSpec((SEL, d), lambda i: (0, 0)),
        ],
        out_specs=[
            pl.BlockSpec((B, d), lambda i: (0, 0)),
            pl.BlockSpec((B, d), lambda i: (0, 0)),
        ],
        out_shape=[
            jax.ShapeDtypeStruct((B, d), F32),
            jax.ShapeDtypeStruct((B, d), F32),
        ],
        scratch_shapes=[pltpu.VMEM((SEL, d), F32)],
    )(parts, parts, xt2, sel_b)


_CH = 112
_RK = 3
_IK = 4


def _sc_aggregate(xt, srcp, dstp, qp, zrows):
    NP, d = xt.shape
    EP = srcp.shape[0]
    CH = _CH
    NCH = EP // CH
    info = plsc.get_sparse_core_info()
    NC, NS = info.num_cores, info.num_subcores
    NW = NC * NS
    CPW = NCH // NW
    ZR = zrows.shape[0]
    RPS = NP // NS
    mesh = plsc.VectorSubcoreMesh(core_axis_name="c", subcore_axis_name="s")

    @functools.partial(
        pl.kernel, mesh=mesh,
        out_type=jax.ShapeDtypeStruct((NC, NP, d), F32),
        scratch_types=[
            pltpu.VMEM((_IK, CH), jnp.int32),
            pltpu.VMEM((_IK, CH), jnp.int32),
            pltpu.VMEM((_IK, CH), jnp.int32),
            pltpu.VMEM((_IK, CH), jnp.int32),
            pltpu.VMEM((_RK, CH, d), F32),
            pltpu.VMEM_SHARED((NP, d), F32),
            pltpu.SemaphoreType.DMA,
            pltpu.SemaphoreType.DMA,
            pltpu.SemaphoreType.DMA,
            pltpu.SemaphoreType.DMA,
            pltpu.SemaphoreType.DMA,
            pltpu.SemaphoreType.DMA,
            pltpu.SemaphoreType.DMA,
        ],
    )
    def agg_kernel(xt_hbm, src_hbm, dst_hbm, qp_hbm, z_hbm, out_hbm,
                   sidx, didx, sidx2, didx2, rows, acc,
                   isem, f0, f1, r0sem, r1sem, gsem, ssem):
        fsems = (f0, f1)
        rsems = (r0sem, r1sem)
        c = lax.axis_index("c")
        s = lax.axis_index("s")
        wid = s * NC + c
        base_chunk = wid * CPW
        row0 = s * RPS
        nfull = RPS // ZR
        rem = RPS - nfull * ZR

        for k in range(nfull):
            pltpu.async_copy(z_hbm, acc.at[pl.ds(row0 + k * ZR, ZR)], isem)
        if rem:
            pltpu.async_copy(z_hbm.at[pl.ds(0, rem)],
                             acc.at[pl.ds(row0 + nfull * ZR, rem)], isem)
        for k in range(nfull):
            pltpu.make_async_copy(z_hbm, acc.at[pl.ds(row0 + k * ZR, ZR)],
                                  isem).wait()
        if rem:
            pltpu.make_async_copy(z_hbm.at[pl.ds(0, rem)],
                                  acc.at[pl.ds(row0 + nfull * ZR, rem)],
                                  isem).wait()
        plsc.subcore_barrier()

        def fetch_cp(k, par):
            eb = (base_chunk + k) * CH
            sl = k % _IK
            sem = fsems[par]
            return (pltpu.make_async_copy(src_hbm.at[pl.ds(eb, CH)],
                                          sidx.at[sl], sem),
                    pltpu.make_async_copy(dst_hbm.at[pl.ds(eb, CH)],
                                          didx.at[sl], sem))

        def remap_cp(k, par):
            sl = k % _IK
            sem = rsems[par]
            return (pltpu.make_async_copy(qp_hbm.at[sidx.at[sl]],
                                          sidx2.at[sl], sem),
                    pltpu.make_async_copy(qp_hbm.at[didx.at[sl]],
                                          didx2.at[sl], sem))

        def gather_cp(k):
            return pltpu.make_async_copy(xt_hbm.at[sidx2.at[k % _IK]],
                                         rows.at[k % _RK], gsem)

        def scatter_cp(k):
            return pltpu.make_async_copy(rows.at[k % _RK],
                                         acc.at[didx2.at[k % _IK]], ssem)

        def start2(cp):
            cp[0].start()
            cp[1].start()

        def wait2(cp):
            cp[0].wait()
            cp[1].wait()

        start2(fetch_cp(0, 0))
        start2(fetch_cp(1, 1))
        wait2(fetch_cp(0, 0))
        start2(remap_cp(0, 0))
        start2(fetch_cp(2, 0))
        wait2(fetch_cp(1, 1))
        start2(remap_cp(1, 1))
        wait2(remap_cp(0, 0))
        gather_cp(0).start()

        def step(i, carry):
            for b in range(2):
                k = 2 * i + b

                @pl.when(k + 2 < CPW)
                def _():
                    wait2(fetch_cp(k + 2, b))

                @pl.when(k + 1 < CPW)
                def _():
                    wait2(remap_cp(k + 1, 1 - b))

                gather_cp(k).wait()

                @pl.when(k > 0)
                def _():
                    scatter_cp(k - 1).wait()

                scatter_cp(k).start(add=True)

                @pl.when(k + 1 < CPW)
                def _():
                    gather_cp(k + 1).start()

                @pl.when(k + 2 < CPW)
                def _():
                    start2(remap_cp(k + 2, b))

                @pl.when(k + 3 < CPW)
                def _():
                    start2(fetch_cp(k + 3, 1 - b))
            return carry

        lax.fori_loop(0, CPW // 2, step, 0)
        scatter_cp(CPW - 1).wait()

        plsc.subcore_barrier()
        pltpu.sync_copy(acc.at[pl.ds(row0, RPS)],
                        out_hbm.at[c, pl.ds(row0, RPS)])

    return agg_kernel(xt, srcp, dstp, qp, zrows)


def kernel(x_1, x_2, n_1, n_2, edge_index, t1_W1, t1_W2, t2_W1, t2_W2,
           g_W1, g_W2, t1_b1, t1_b2, t2_b1, t2_b2, g_b1, g_b2):
    N1, d = x_1.shape
    N2 = x_2.shape[0]
    N = N1 + N2
    B = n_1.shape[0]
    P1 = -(-N1 // 128) * 128
    P2 = -(-N2 // 128) * 128
    NP = P1 + P2

    n1 = n_1.astype(jnp.int32)
    n2 = n_2.astype(jnp.int32)
    cum = jnp.cumsum(n1 + n2)
    zero = jnp.zeros((1,), jnp.int32)
    C0 = jnp.concatenate([zero, cum[:-1]])
    c1 = jnp.concatenate([zero, jnp.cumsum(n1)[:-1]])
    c2 = jnp.concatenate([zero, jnp.cumsum(n2)[:-1]])
    j = jnp.arange(N, dtype=jnp.int32)
    g = jnp.searchsorted(cum, j, side='right')
    within = j - C0[g]
    perm = jnp.where(within < n1[g], c1[g] + within,
                     N1 + c2[g] + within - n1[g]).astype(jnp.int32)
    qp = perm + jnp.where(perm >= N1, P1 - N1, 0).astype(jnp.int32)
    QPAD = -(-(N + 1) // 16) * 16
    qp_pad = jnp.concatenate([qp, jnp.full((QPAD - N,), NP - 1, jnp.int32)])
    a_idx = C0
    b_idx = C0 + n1
    SEL = -(-(2 * B) // 128) * 128
    sel = jnp.concatenate([qp[a_idx], qp[b_idx],
                           jnp.zeros((SEL - 2 * B,), jnp.int32)])
    sel_b = jnp.broadcast_to(sel[:, None], (SEL, d))

    E = edge_index.shape[1]
    CH = _CH
    info = plsc.get_sparse_core_info()
    NW = info.num_cores * info.num_subcores
    GRAN = CH * NW * 2
    EP = -(-E // GRAN) * GRAN
    src = edge_index[0].astype(jnp.int32)
    dst = edge_index[1].astype(jnp.int32)
    if EP != E:
        src = jnp.concatenate([src, jnp.zeros((EP - E,), jnp.int32)])
        dst = jnp.concatenate([dst, jnp.full((EP - E,), N, jnp.int32)])

    zrows = jnp.zeros((CH, d), F32)

    Xs = jnp.stack([jnp.pad(x_1, ((0, P1 - N1), (0, 0))),
                    jnp.pad(x_2, ((0, P2 - N2), (0, 0)))])
    TW1 = jnp.stack([t1_W1, t2_W1])
    TW2 = jnp.stack([t1_W2, t2_W2])
    Tb1 = jnp.stack([t1_b1, t2_b1]).reshape(2, 1, d)
    Tb2 = jnp.stack([t1_b2, t2_b2]).reshape(2, 1, d)
    gb1 = g_b1.reshape(1, d)
    gb2 = g_b2.reshape(1, d)

    xt1 = _run_stage_a(Xs, TW1, Tb1, TW2, Tb2, g_W1, gb1, NP)
    parts1 = _sc_aggregate(xt1, src, dst, qp_pad, zrows)
    xt2 = _run_stage_d(parts1, xt1, g_W2, gb2)
    parts2 = _sc_aggregate(xt2, src, dst, qp_pad, zrows)
    o1, o2 = _run_stage_f(parts2, xt2, sel_b.astype(jnp.int32), B)
    return (o1, o2)


# speedup vs baseline: 1.2112x; 1.2112x over previous
import functools

import jax
import jax.numpy as jnp
from jax import lax
from jax.experimental import pallas as pl
from jax.experimental.pallas import tpu as pltpu
from jax.experimental.pallas import tpu_sc as plsc

F32 = jnp.float32
_INV_S = 1.0 / (1.0 + 1e-5) ** 0.5
_MAXN = 1.0 - 1e-5
_PREC = lax.Precision.HIGHEST


def _rnorm(x):
    return jnp.maximum(jnp.sqrt(jnp.sum(x * x, axis=-1, keepdims=True)), 1e-15)


def _artanh(x):
    x = jnp.clip(x, -1.0 + 1e-7, 1.0 - 1e-7)
    return 0.5 * (jnp.log1p(x) - jnp.log1p(-x))


def _proj(x):
    n = _rnorm(x)
    return jnp.where(n > _MAXN, x / n * _MAXN, x)


def _expmap0(u):
    n = _rnorm(u)
    return jnp.tanh(n) * u / n


def _logmap0(p):
    n = _rnorm(p)
    return _artanh(n) * p / n


def _mobius_add(x, y):
    x2 = jnp.sum(x * x, -1, keepdims=True)
    y2 = jnp.sum(y * y, -1, keepdims=True)
    xy = jnp.sum(x * y, -1, keepdims=True)
    num = (1 + 2 * xy + y2) * x + (1 - x2) * y
    den = 1 + 2 * xy + x2 * y2
    return num / jnp.maximum(den, 1e-15)


def _matvecT(x, W):
    return lax.dot_general(x, W, (((1,), (1,)), ((), ())),
                           precision=_PREC, preferred_element_type=F32)


def _mobius_matvec_block(y, W):
    xn = _rnorm(y)
    mx = _matvecT(y, W)
    mn = _rnorm(mx)
    return jnp.tanh(mn / xn * _artanh(xn)) * mx / mn


def _stage_a_body(x_ref, w1_ref, b1_ref, w2_ref, b2_ref, gw_ref, gb_ref,
                  out_ref):
    x = x_ref[0]
    h = jax.nn.relu((_matvecT(x, w1_ref[0]) + b1_ref[0]) * _INV_S)
    h = jax.nn.relu((_matvecT(h, w2_ref[0]) + b2_ref[0]) * _INV_S)
    y = _proj(_expmap0(h))
    mv = _proj(_mobius_matvec_block(y, gw_ref[...]))
    gb = _proj(_expmap0(gb_ref[...]))
    h2 = _proj(_mobius_add(mv, gb))
    out_ref[...] = _logmap0(h2)


def _run_stage_a(Xs, TW1, Tb1, TW2, Tb2, gW, gb, NP):
    T, P, d = Xs.shape
    PB = P // 128
    grid = (T, PB)
    return pl.pallas_call(
        _stage_a_body,
        grid=grid,
        in_specs=[
            pl.BlockSpec((1, 128, d), lambda t, i: (t, i, 0)),
            pl.BlockSpec((1, d, d), lambda t, i: (t, 0, 0)),
            pl.BlockSpec((1, 1, d), lambda t, i: (t, 0, 0)),
            pl.BlockSpec((1, d, d), lambda t, i: (t, 0, 0)),
            pl.BlockSpec((1, 1, d), lambda t, i: (t, 0, 0)),
            pl.BlockSpec((d, d), lambda t, i: (0, 0)),
            pl.BlockSpec((1, d), lambda t, i: (0, 0)),
        ],
        out_specs=pl.BlockSpec((128, d), lambda t, i: (t * PB + i, 0)),
        out_shape=jax.ShapeDtypeStruct((NP, d), F32),
    )(Xs, TW1, Tb1, TW2, Tb2, gW, gb)


def _stage_d_body(p0_ref, p1_ref, xt_ref, gw_ref, gb_ref, out_ref):
    agg = p0_ref[0] + p1_ref[0] + xt_ref[...]
    h = _proj(_expmap0(agg))
    h = _proj(_expmap0(jax.nn.relu(_logmap0(h))))
    mv = _proj(_mobius_matvec_block(h, gw_ref[...]))
    gb = _proj(_expmap0(gb_ref[...]))
    out_ref[...] = _logmap0(_proj(_mobius_add(mv, gb)))


def _run_stage_d(parts, xt1, gW, gb):
    NP, d = xt1.shape
    grid = (NP // 128,)
    return pl.pallas_call(
        _stage_d_body,
        grid=grid,
        in_specs=[
            pl.BlockSpec((1, 128, d), lambda i: (0, i, 0)),
            pl.BlockSpec((1, 128, d), lambda i: (1, i, 0)),
            pl.BlockSpec((128, d), lambda i: (i, 0)),
            pl.BlockSpec((d, d), lambda i: (0, 0)),
            pl.BlockSpec((1, d), lambda i: (0, 0)),
        ],
        out_specs=pl.BlockSpec((128, d), lambda i: (i, 0)),
        out_shape=jax.ShapeDtypeStruct((NP, d), F32),
    )(parts, parts, xt1, gW, gb)


def _stage_f_body(p0_ref, p1_ref, xt_ref, sel_ref, o1_ref, o2_ref, acc_ref):
    i = pl.program_id(0)
    SEL = acc_ref.shape[0]
    B = o1_ref.shape[0]

    @pl.when(i == 0)
    def _():
        acc_ref[...] = jnp.zeros_like(acc_ref)

    blk = p0_ref[0] + p1_ref[0] + xt_ref[...]
    col = lax.broadcasted_iota(jnp.int32, (SEL, 128), 1) + i * 128
    oh = (sel_ref[...] == col).astype(F32)
    acc_ref[...] += lax.dot_general(oh, blk, (((1,), (0,)), ((), ())),
                                    precision=_PREC,
                                    preferred_element_type=F32)

    @pl.when(i == pl.num_programs(0) - 1)
    def _():
        agg = acc_ref[...]
        h = _proj(_expmap0(agg))
        h = _proj(_expmap0(jax.nn.relu(_logmap0(h))))
        res = _proj(_expmap0(_logmap0(h)))
        o1_ref[...] = res[:B]
        o2_ref[...] = res[B:2 * B]


def _run_stage_f(parts, xt2, sel_b, B):
    NP, d = xt2.shape
    SEL = sel_b.shape[0]
    grid = (NP // 128,)
    return pl.pallas_call(
        _stage_f_body,
        grid=grid,
        in_specs=[
            pl.BlockSpec((1, 128, d), lambda i: (0, i, 0)),
            pl.BlockSpec((1, 128, d), lambda i: (1, i, 0)),
            pl.BlockSpec((128, d), lambda i: (i, 0)),
            pl.BlockSpec((SEL, d), lambda i: (0, 0)),
        ],
        out_specs=[
            pl.BlockSpec((B, d), lambda i: (0, 0)),
            pl.BlockSpec((B, d), lambda i: (0, 0)),
        ],
        out_shape=[
            jax.ShapeDtypeStruct((B, d), F32),
            jax.ShapeDtypeStruct((B, d), F32),
        ],
        scratch_shapes=[pltpu.VMEM((SEL, d), F32)],
    )(parts, parts, xt2, sel_b)


_CH = 128


def _sc_aggregate(xt, srcp, dstp, qp, zrows):
    NP, d = xt.shape
    EP = srcp.shape[0]
    CH = _CH
    NCH = EP // CH
    info = plsc.get_sparse_core_info()
    NC, NS = info.num_cores, info.num_subcores
    NW = NC * NS
    CPW = NCH // NW
    ZR = zrows.shape[0]
    RPS = NP // NS
    mesh = plsc.VectorSubcoreMesh(core_axis_name="c", subcore_axis_name="s")

    @functools.partial(
        pl.kernel, mesh=mesh,
        out_type=jax.ShapeDtypeStruct((NC, NP, d), F32),
        scratch_types=[
            pltpu.VMEM((2, CH), jnp.int32),
            pltpu.VMEM((2, CH), jnp.int32),
            pltpu.VMEM((2, CH), jnp.int32),
            pltpu.VMEM((3, CH), jnp.int32),
            pltpu.VMEM((3, CH, d), F32),
            pltpu.VMEM_SHARED((NP, d), F32),
            pltpu.SemaphoreType.DMA,
            pltpu.SemaphoreType.DMA,
            pltpu.SemaphoreType.DMA,
            pltpu.SemaphoreType.DMA,
        ],
    )
    def agg_kernel(xt_hbm, src_hbm, dst_hbm, qp_hbm, z_hbm, out_hbm,
                   sidx, didx, sidx2, didx2, rows, acc,
                   isem, gsem, ss0, ss1):
        ssems = (ss0, ss1)
        c = lax.axis_index("c")
        s = lax.axis_index("s")
        wid = s * NC + c
        base_chunk = wid * CPW
        row0 = s * RPS
        nfull = RPS // ZR
        rem = RPS - nfull * ZR

        for k in range(nfull):
            pltpu.async_copy(z_hbm, acc.at[pl.ds(row0 + k * ZR, ZR)], isem)
        if rem:
            pltpu.async_copy(z_hbm.at[pl.ds(0, rem)],
                             acc.at[pl.ds(row0 + nfull * ZR, rem)], isem)
        for k in range(nfull):
            pltpu.make_async_copy(z_hbm, acc.at[pl.ds(row0 + k * ZR, ZR)],
                                  isem).wait()
        if rem:
            pltpu.make_async_copy(z_hbm.at[pl.ds(0, rem)],
                                  acc.at[pl.ds(row0 + nfull * ZR, rem)],
                                  isem).wait()
        plsc.subcore_barrier()

        def fetch_remap(k, p):
            eb = (base_chunk + k) * CH
            pltpu.sync_copy(src_hbm.at[pl.ds(eb, CH)], sidx.at[p])
            pltpu.sync_copy(dst_hbm.at[pl.ds(eb, CH)], didx.at[p])
            cp_s = pltpu.async_copy(qp_hbm.at[sidx.at[p]], sidx2.at[p],
                                    isem)
            cp_d = pltpu.async_copy(qp_hbm.at[didx.at[p]], didx2.at[k % 3],
                                    isem)
            cp_s.wait()
            cp_d.wait()

        def gather_cp(k, p):
            return pltpu.make_async_copy(xt_hbm.at[sidx2.at[p]],
                                         rows.at[k % 3], gsem)

        def scatter_cp(k, p):
            return pltpu.make_async_copy(rows.at[k % 3],
                                         acc.at[didx2.at[k % 3]], ssems[p])

        fetch_remap(0, 0)
        gather_cp(0, 0).start()

        def step(i, carry):
            for b in range(2):
                k = 2 * i + b
                nb = 1 - b

                @pl.when(k > 1)
                def _():
                    scatter_cp(k - 2, b).wait()

                @pl.when(k + 1 < CPW)
                def _():
                    fetch_remap(k + 1, nb)

                gather_cp(k, b).wait()

                @pl.when(k + 1 < CPW)
                def _():
                    gather_cp(k + 1, nb).start()

                scatter_cp(k, b).start(add=True)
            return carry

        lax.fori_loop(0, CPW // 2, step, 0)
        scatter_cp(CPW - 2, (CPW - 2) % 2).wait()
        scatter_cp(CPW - 1, (CPW - 1) % 2).wait()

        plsc.subcore_barrier()
        pltpu.sync_copy(acc.at[pl.ds(row0, RPS)],
                        out_hbm.at[c, pl.ds(row0, RPS)])

    return agg_kernel(xt, srcp, dstp, qp, zrows)


def kernel(x_1, x_2, n_1, n_2, edge_index, t1_W1, t1_W2, t2_W1, t2_W2,
           g_W1, g_W2, t1_b1, t1_b2, t2_b1, t2_b2, g_b1, g_b2):
    N1, d = x_1.shape
    N2 = x_2.shape[0]
    N = N1 + N2
    B = n_1.shape[0]
    P1 = -(-N1 // 128) * 128
    P2 = -(-N2 // 128) * 128
    NP = P1 + P2

    n1 = n_1.astype(jnp.int32)
    n2 = n_2.astype(jnp.int32)
    cum = jnp.cumsum(n1 + n2)
    zero = jnp.zeros((1,), jnp.int32)
    C0 = jnp.concatenate([zero, cum[:-1]])
    c1 = jnp.concatenate([zero, jnp.cumsum(n1)[:-1]])
    c2 = jnp.concatenate([zero, jnp.cumsum(n2)[:-1]])
    j = jnp.arange(N, dtype=jnp.int32)
    g = jnp.searchsorted(cum, j, side='right')
    within = j - C0[g]
    perm = jnp.where(within < n1[g], c1[g] + within,
                     N1 + c2[g] + within - n1[g]).astype(jnp.int32)
    qp = perm + jnp.where(perm >= N1, P1 - N1, 0).astype(jnp.int32)
    QPAD = -(-(N + 1) // 16) * 16
    qp_pad = jnp.concatenate([qp, jnp.full((QPAD - N,), NP - 1, jnp.int32)])
    a_idx = C0
    b_idx = C0 + n1
    SEL = -(-(2 * B) // 128) * 128
    sel = jnp.concatenate([qp[a_idx], qp[b_idx],
                           jnp.zeros((SEL - 2 * B,), jnp.int32)])
    sel_b = jnp.broadcast_to(sel[:, None], (SEL, d))

    E = edge_index.shape[1]
    CH = _CH
    info = plsc.get_sparse_core_info()
    NW = info.num_cores * info.num_subcores
    GRAN = CH * NW * 2
    EP = -(-E // GRAN) * GRAN
    src = edge_index[0].astype(jnp.int32)
    dst = edge_index[1].astype(jnp.int32)
    if EP != E:
        src = jnp.concatenate([src, jnp.zeros((EP - E,), jnp.int32)])
        dst = jnp.concatenate([dst, jnp.full((EP - E,), N, jnp.int32)])

    zrows = jnp.zeros((CH, d), F32)

    Xs = jnp.stack([jnp.pad(x_1, ((0, P1 - N1), (0, 0))),
                    jnp.pad(x_2, ((0, P2 - N2), (0, 0)))])
    TW1 = jnp.stack([t1_W1, t2_W1])
    TW2 = jnp.stack([t1_W2, t2_W2])
    Tb1 = jnp.stack([t1_b1, t2_b1]).reshape(2, 1, d)
    Tb2 = jnp.stack([t1_b2, t2_b2]).reshape(2, 1, d)
    gb1 = g_b1.reshape(1, d)
    gb2 = g_b2.reshape(1, d)

    xt1 = _run_stage_a(Xs, TW1, Tb1, TW2, Tb2, g_W1, gb1, NP)
    parts1 = _sc_aggregate(xt1, src, dst, qp_pad, zrows)
    xt2 = _run_stage_d(parts1, xt1, g_W2, gb2)
    parts2 = _sc_aggregate(xt2, src, dst, qp_pad, zrows)
    o1, o2 = _run_stage_f(parts2, xt2, sel_b.astype(jnp.int32), B)
    return (o1, o2)


# speedup vs baseline: 1.9096x; 1.5766x over previous
import functools

import jax
import jax.numpy as jnp
from jax import lax
from jax.experimental import pallas as pl
from jax.experimental.pallas import tpu as pltpu
from jax.experimental.pallas import tpu_sc as plsc

F32 = jnp.float32
_INV_S = 1.0 / (1.0 + 1e-5) ** 0.5
_MAXN = 1.0 - 1e-5
_PREC = lax.Precision.HIGHEST


def _rnorm(x):
    return jnp.maximum(jnp.sqrt(jnp.sum(x * x, axis=-1, keepdims=True)), 1e-15)


def _artanh(x):
    x = jnp.clip(x, -1.0 + 1e-7, 1.0 - 1e-7)
    return 0.5 * (jnp.log1p(x) - jnp.log1p(-x))


def _proj(x):
    n = _rnorm(x)
    return jnp.where(n > _MAXN, x / n * _MAXN, x)


def _expmap0(u):
    n = _rnorm(u)
    return jnp.tanh(n) * u / n


def _logmap0(p):
    n = _rnorm(p)
    return _artanh(n) * p / n


def _mobius_add(x, y):
    x2 = jnp.sum(x * x, -1, keepdims=True)
    y2 = jnp.sum(y * y, -1, keepdims=True)
    xy = jnp.sum(x * y, -1, keepdims=True)
    num = (1 + 2 * xy + y2) * x + (1 - x2) * y
    den = 1 + 2 * xy + x2 * y2
    return num / jnp.maximum(den, 1e-15)


def _matvecT(x, W):
    return lax.dot_general(x, W, (((1,), (1,)), ((), ())),
                           precision=_PREC, preferred_element_type=F32)


def _mobius_matvec_block(y, W):
    xn = _rnorm(y)
    mx = _matvecT(y, W)
    mn = _rnorm(mx)
    return jnp.tanh(mn / xn * _artanh(xn)) * mx / mn


def _stage_a_body(x_ref, w1_ref, b1_ref, w2_ref, b2_ref, gw_ref, gb_ref,
                  out_ref):
    x = x_ref[0]
    h = jax.nn.relu((_matvecT(x, w1_ref[0]) + b1_ref[0]) * _INV_S)
    h = jax.nn.relu((_matvecT(h, w2_ref[0]) + b2_ref[0]) * _INV_S)
    y = _proj(_expmap0(h))
    mv = _proj(_mobius_matvec_block(y, gw_ref[...]))
    gb = _proj(_expmap0(gb_ref[...]))
    h2 = _proj(_mobius_add(mv, gb))
    out_ref[...] = _logmap0(h2)


def _run_stage_a(Xs, TW1, Tb1, TW2, Tb2, gW, gb, NP):
    T, P, d = Xs.shape
    PB = P // 128
    grid = (T, PB)
    return pl.pallas_call(
        _stage_a_body,
        grid=grid,
        in_specs=[
            pl.BlockSpec((1, 128, d), lambda t, i: (t, i, 0)),
            pl.BlockSpec((1, d, d), lambda t, i: (t, 0, 0)),
            pl.BlockSpec((1, 1, d), lambda t, i: (t, 0, 0)),
            pl.BlockSpec((1, d, d), lambda t, i: (t, 0, 0)),
            pl.BlockSpec((1, 1, d), lambda t, i: (t, 0, 0)),
            pl.BlockSpec((d, d), lambda t, i: (0, 0)),
            pl.BlockSpec((1, d), lambda t, i: (0, 0)),
        ],
        out_specs=pl.BlockSpec((128, d), lambda t, i: (t * PB + i, 0)),
        out_shape=jax.ShapeDtypeStruct((NP, d), F32),
    )(Xs, TW1, Tb1, TW2, Tb2, gW, gb)


def _stage_d_body(p0_ref, p1_ref, xt_ref, gw_ref, gb_ref, out_ref):
    agg = p0_ref[0] + p1_ref[0] + xt_ref[...]
    h = _proj(_expmap0(agg))
    h = _proj(_expmap0(jax.nn.relu(_logmap0(h))))
    mv = _proj(_mobius_matvec_block(h, gw_ref[...]))
    gb = _proj(_expmap0(gb_ref[...]))
    out_ref[...] = _logmap0(_proj(_mobius_add(mv, gb)))


def _run_stage_d(parts, xt1, gW, gb):
    NP, d = xt1.shape
    grid = (NP // 128,)
    return pl.pallas_call(
        _stage_d_body,
        grid=grid,
        in_specs=[
            pl.BlockSpec((1, 128, d), lambda i: (0, i, 0)),
            pl.BlockSpec((1, 128, d), lambda i: (1, i, 0)),
            pl.BlockSpec((128, d), lambda i: (i, 0)),
            pl.BlockSpec((d, d), lambda i: (0, 0)),
            pl.BlockSpec((1, d), lambda i: (0, 0)),
        ],
        out_specs=pl.BlockSpec((128, d), lambda i: (i, 0)),
        out_shape=jax.ShapeDtypeStruct((NP, d), F32),
    )(parts, parts, xt1, gW, gb)


def _stage_f_body(p0_ref, p1_ref, xt_ref, sel_ref, o1_ref, o2_ref, acc_ref):
    i = pl.program_id(0)
    SEL = acc_ref.shape[0]
    B = o1_ref.shape[0]

    @pl.when(i == 0)
    def _():
        acc_ref[...] = jnp.zeros_like(acc_ref)

    blk = p0_ref[0] + p1_ref[0] + xt_ref[...]
    col = lax.broadcasted_iota(jnp.int32, (SEL, 128), 1) + i * 128
    oh = (sel_ref[...] == col).astype(F32)
    acc_ref[...] += lax.dot_general(oh, blk, (((1,), (0,)), ((), ())),
                                    precision=_PREC,
                                    preferred_element_type=F32)

    @pl.when(i == pl.num_programs(0) - 1)
    def _():
        agg = acc_ref[...]
        h = _proj(_expmap0(agg))
        h = _proj(_expmap0(jax.nn.relu(_logmap0(h))))
        res = _proj(_expmap0(_logmap0(h)))
        o1_ref[...] = res[:B]
        o2_ref[...] = res[B:2 * B]


def _run_stage_f(parts, xt2, sel_b, B):
    NP, d = xt2.shape
    SEL = sel_b.shape[0]
    grid = (NP // 128,)
    return pl.pallas_call(
        _stage_f_body,
        grid=grid,
        in_specs=[
            pl.BlockSpec((1, 128, d), lambda i: (0, i, 0)),
            pl.BlockSpec((1, 128, d), lambda i: (1, i, 0)),
            pl.BlockSpec((128, d), lambda i: (i, 0)),
            pl.BlockSpec((SEL, d), lambda i: (0, 0)),
        ],
        out_specs=[
            pl.BlockSpec((B, d), lambda i: (0, 0)),
            pl.BlockSpec((B, d), lambda i: (0, 0)),
        ],
        out_shape=[
            jax.ShapeDtypeStruct((B, d), F32),
            jax.ShapeDtypeStruct((B, d), F32),
        ],
        scratch_shapes=[pltpu.VMEM((SEL, d), F32)],
    )(parts, parts, xt2, sel_b)


_CH = 128


def _sc_aggregate(xt, srcp, dstp, qp, zrows):
    NP, d = xt.shape
    EP = srcp.shape[0]
    CH = _CH
    NCH = EP // CH
    info = plsc.get_sparse_core_info()
    NC, NS = info.num_cores, info.num_subcores
    NW = NC * NS
    CPW = NCH // NW
    ZR = zrows.shape[0]
    RPS = NP // NS
    mesh = plsc.VectorSubcoreMesh(core_axis_name="c", subcore_axis_name="s")

    @functools.partial(
        pl.kernel, mesh=mesh,
        out_type=jax.ShapeDtypeStruct((NC, NP, d), F32),
        scratch_types=[
            pltpu.VMEM((2, CH), jnp.int32),
            pltpu.VMEM((2, CH), jnp.int32),
            pltpu.VMEM((2, CH), jnp.int32),
            pltpu.VMEM((3, CH), jnp.int32),
            pltpu.VMEM((3, CH, d), F32),
            pltpu.VMEM_SHARED((NP, d), F32),
            pltpu.SemaphoreType.DMA,
            pltpu.SemaphoreType.DMA,
            pltpu.SemaphoreType.DMA,
            pltpu.SemaphoreType.DMA,
        ],
    )
    def agg_kernel(xt_hbm, src_hbm, dst_hbm, qp_hbm, z_hbm, out_hbm,
                   sidx, didx, sidx2, didx2, rows, acc,
                   isem, gsem, ss0, ss1):
        ssems = (ss0, ss1)
        c = lax.axis_index("c")
        s = lax.axis_index("s")
        wid = s * NC + c
        base_chunk = wid * CPW
        row0 = s * RPS
        nfull = RPS // ZR
        rem = RPS - nfull * ZR

        for k in range(nfull):
            pltpu.async_copy(z_hbm, acc.at[pl.ds(row0 + k * ZR, ZR)], isem)
        if rem:
            pltpu.async_copy(z_hbm.at[pl.ds(0, rem)],
                             acc.at[pl.ds(row0 + nfull * ZR, rem)], isem)
        for k in range(nfull):
            pltpu.make_async_copy(z_hbm, acc.at[pl.ds(row0 + k * ZR, ZR)],
                                  isem).wait()
        if rem:
            pltpu.make_async_copy(z_hbm.at[pl.ds(0, rem)],
                                  acc.at[pl.ds(row0 + nfull * ZR, rem)],
                                  isem).wait()
        plsc.subcore_barrier()

        def fetch_remap(k, p):
            eb = (base_chunk + k) * CH
            pltpu.sync_copy(src_hbm.at[pl.ds(eb, CH)], sidx.at[p])
            pltpu.sync_copy(dst_hbm.at[pl.ds(eb, CH)], didx.at[p])
            cp_s = pltpu.async_copy(qp_hbm.at[sidx.at[p]], sidx2.at[p],
                                    isem)
            cp_d = pltpu.async_copy(qp_hbm.at[didx.at[p]], didx2.at[k % 3],
                                    isem)
            cp_s.wait()
            cp_d.wait()

        def gather_cp(k, p):
            return pltpu.make_async_copy(xt_hbm.at[sidx2.at[p]],
                                         rows.at[k % 3], gsem)

        def scatter_cp(k, p):
            return pltpu.make_async_copy(rows.at[k % 3],
                                         acc.at[didx2.at[k % 3]], ssems[p])

        fetch_remap(0, 0)
        gather_cp(0, 0).start()

        def step(i, carry):
            for b in range(2):
                k = 2 * i + b
                nb = 1 - b

                @pl.when(k > 1)
                def _():
                    scatter_cp(k - 2, b).wait()

                @pl.when(k + 1 < CPW)
                def _():
                    fetch_remap(k + 1, nb)

                gather_cp(k, b).wait()

                @pl.when(k + 1 < CPW)
                def _():
                    gather_cp(k + 1, nb).start()

                scatter_cp(k, b).start(add=True)
            return carry

        lax.fori_loop(0, CPW // 2, step, 0)
        scatter_cp(CPW - 2, (CPW - 2) % 2).wait()
        scatter_cp(CPW - 1, (CPW - 1) % 2).wait()

        plsc.subcore_barrier()
        pltpu.sync_copy(acc.at[pl.ds(row0, RPS)],
                        out_hbm.at[c, pl.ds(row0, RPS)])

    return agg_kernel(xt, srcp, dstp, qp, zrows)


def kernel(x_1, x_2, n_1, n_2, edge_index, t1_W1, t1_W2, t2_W1, t2_W2,
           g_W1, g_W2, t1_b1, t1_b2, t2_b1, t2_b2, g_b1, g_b2):
    N1, d = x_1.shape
    N2 = x_2.shape[0]
    N = N1 + N2
    B = n_1.shape[0]
    P1 = -(-N1 // 128) * 128
    P2 = -(-N2 // 128) * 128
    NP = P1 + P2

    n1 = n_1.astype(jnp.int32)
    n2 = n_2.astype(jnp.int32)
    cum = jnp.cumsum(n1 + n2)
    zero = jnp.zeros((1,), jnp.int32)
    C0 = jnp.concatenate([zero, cum[:-1]])
    c1 = jnp.concatenate([zero, jnp.cumsum(n1)[:-1]])
    c2 = jnp.concatenate([zero, jnp.cumsum(n2)[:-1]])
    j = jnp.arange(N, dtype=jnp.int32)
    geB = (j[:, None] >= cum[None, :]).astype(F32)
    ge_ext = jnp.concatenate([jnp.ones((N, 1), F32), geB], axis=1)
    oh = ge_ext[:, :B] - ge_ext[:, 1:B + 1]
    tbl = jnp.stack([C0, n1, c1, c2], axis=1).astype(F32)
    vals = jax.lax.dot_general(oh, tbl, (((1,), (0,)), ((), ())),
                               precision=_PREC)
    C0g = vals[:, 0].astype(jnp.int32)
    n1g = vals[:, 1].astype(jnp.int32)
    c1g = vals[:, 2].astype(jnp.int32)
    c2g = vals[:, 3].astype(jnp.int32)
    within = j - C0g
    perm = jnp.where(within < n1g, c1g + within,
                     N1 + c2g + within - n1g).astype(jnp.int32)
    qp = perm + jnp.where(perm >= N1, P1 - N1, 0).astype(jnp.int32)
    QPAD = -(-(N + 1) // 16) * 16
    qp_pad = jnp.concatenate([qp, jnp.full((QPAD - N,), NP - 1, jnp.int32)])
    a_idx = C0
    b_idx = C0 + n1
    SEL = -(-(2 * B) // 128) * 128
    sel = jnp.concatenate([qp[a_idx], qp[b_idx],
                           jnp.zeros((SEL - 2 * B,), jnp.int32)])
    sel_b = jnp.broadcast_to(sel[:, None], (SEL, d))

    E = edge_index.shape[1]
    CH = _CH
    info = plsc.get_sparse_core_info()
    NW = info.num_cores * info.num_subcores
    GRAN = CH * NW * 2
    EP = -(-E // GRAN) * GRAN
    src = edge_index[0].astype(jnp.int32)
    dst = edge_index[1].astype(jnp.int32)
    if EP != E:
        src = jnp.concatenate([src, jnp.zeros((EP - E,), jnp.int32)])
        dst = jnp.concatenate([dst, jnp.full((EP - E,), N, jnp.int32)])

    zrows = jnp.zeros((CH, d), F32)

    Xs = jnp.stack([jnp.pad(x_1, ((0, P1 - N1), (0, 0))),
                    jnp.pad(x_2, ((0, P2 - N2), (0, 0)))])
    TW1 = jnp.stack([t1_W1, t2_W1])
    TW2 = jnp.stack([t1_W2, t2_W2])
    Tb1 = jnp.stack([t1_b1, t2_b1]).reshape(2, 1, d)
    Tb2 = jnp.stack([t1_b2, t2_b2]).reshape(2, 1, d)
    gb1 = g_b1.reshape(1, d)
    gb2 = g_b2.reshape(1, d)

    xt1 = _run_stage_a(Xs, TW1, Tb1, TW2, Tb2, g_W1, gb1, NP)
    parts1 = _sc_aggregate(xt1, src, dst, qp_pad, zrows)
    xt2 = _run_stage_d(parts1, xt1, g_W2, gb2)
    parts2 = _sc_aggregate(xt2, src, dst, qp_pad, zrows)
    o1, o2 = _run_stage_f(parts2, xt2, sel_b.astype(jnp.int32), B)
    return (o1, o2)


# speedup vs baseline: 2.3780x; 1.2453x over previous
import functools

import jax
import jax.numpy as jnp
from jax import lax
from jax.experimental import pallas as pl
from jax.experimental.pallas import tpu as pltpu
from jax.experimental.pallas import tpu_sc as plsc

F32 = jnp.float32
_INV_S = 1.0 / (1.0 + 1e-5) ** 0.5
_MAXN = 1.0 - 1e-5
_PREC = lax.Precision.HIGHEST


def _rnorm(x):
    return jnp.maximum(jnp.sqrt(jnp.sum(x * x, axis=-1, keepdims=True)), 1e-15)


def _artanh(x):
    x = jnp.clip(x, -1.0 + 1e-7, 1.0 - 1e-7)
    return 0.5 * (jnp.log1p(x) - jnp.log1p(-x))


def _proj(x):
    n = _rnorm(x)
    return jnp.where(n > _MAXN, x / n * _MAXN, x)


def _expmap0(u):
    n = _rnorm(u)
    return jnp.tanh(n) * u / n


def _logmap0(p):
    n = _rnorm(p)
    return _artanh(n) * p / n


def _mobius_add(x, y):
    x2 = jnp.sum(x * x, -1, keepdims=True)
    y2 = jnp.sum(y * y, -1, keepdims=True)
    xy = jnp.sum(x * y, -1, keepdims=True)
    num = (1 + 2 * xy + y2) * x + (1 - x2) * y
    den = 1 + 2 * xy + x2 * y2
    return num / jnp.maximum(den, 1e-15)


def _matvecT(x, W):
    return lax.dot_general(x, W, (((1,), (1,)), ((), ())),
                           precision=_PREC, preferred_element_type=F32)


def _mobius_matvec_block(y, W):
    xn = _rnorm(y)
    mx = _matvecT(y, W)
    mn = _rnorm(mx)
    return jnp.tanh(mn / xn * _artanh(xn)) * mx / mn


def _stage_a_body(x_ref, w1_ref, b1_ref, w2_ref, b2_ref, gw_ref, gb_ref,
                  out_ref):
    x = x_ref[0]
    h = jax.nn.relu((_matvecT(x, w1_ref[0]) + b1_ref[0]) * _INV_S)
    h = jax.nn.relu((_matvecT(h, w2_ref[0]) + b2_ref[0]) * _INV_S)
    y = _proj(_expmap0(h))
    mv = _proj(_mobius_matvec_block(y, gw_ref[...]))
    gb = _proj(_expmap0(gb_ref[...]))
    h2 = _proj(_mobius_add(mv, gb))
    out_ref[...] = _logmap0(h2)


def _run_stage_a(Xs, TW1, Tb1, TW2, Tb2, gW, gb, NP):
    T, P, d = Xs.shape
    PB = P // 128
    grid = (T, PB)
    return pl.pallas_call(
        _stage_a_body,
        grid=grid,
        in_specs=[
            pl.BlockSpec((1, 128, d), lambda t, i: (t, i, 0)),
            pl.BlockSpec((1, d, d), lambda t, i: (t, 0, 0)),
            pl.BlockSpec((1, 1, d), lambda t, i: (t, 0, 0)),
            pl.BlockSpec((1, d, d), lambda t, i: (t, 0, 0)),
            pl.BlockSpec((1, 1, d), lambda t, i: (t, 0, 0)),
            pl.BlockSpec((d, d), lambda t, i: (0, 0)),
            pl.BlockSpec((1, d), lambda t, i: (0, 0)),
        ],
        out_specs=pl.BlockSpec((128, d), lambda t, i: (t * PB + i, 0)),
        out_shape=jax.ShapeDtypeStruct((NP, d), F32),
    )(Xs, TW1, Tb1, TW2, Tb2, gW, gb)


def _stage_d_body(p0_ref, p1_ref, xt_ref, gw_ref, gb_ref, out_ref):
    agg = p0_ref[0] + p1_ref[0] + xt_ref[...]
    h = _proj(_expmap0(agg))
    h = _proj(_expmap0(jax.nn.relu(_logmap0(h))))
    mv = _proj(_mobius_matvec_block(h, gw_ref[...]))
    gb = _proj(_expmap0(gb_ref[...]))
    out_ref[...] = _logmap0(_proj(_mobius_add(mv, gb)))


def _run_stage_d(parts, xt1, gW, gb):
    NP, d = xt1.shape
    grid = (NP // 128,)
    return pl.pallas_call(
        _stage_d_body,
        grid=grid,
        in_specs=[
            pl.BlockSpec((1, 128, d), lambda i: (0, i, 0)),
            pl.BlockSpec((1, 128, d), lambda i: (1, i, 0)),
            pl.BlockSpec((128, d), lambda i: (i, 0)),
            pl.BlockSpec((d, d), lambda i: (0, 0)),
            pl.BlockSpec((1, d), lambda i: (0, 0)),
        ],
        out_specs=pl.BlockSpec((128, d), lambda i: (i, 0)),
        out_shape=jax.ShapeDtypeStruct((NP, d), F32),
    )(parts, parts, xt1, gW, gb)


def _stage_f_body(p0_ref, p1_ref, xt_ref, sel_ref, o1_ref, o2_ref, acc_ref):
    i = pl.program_id(0)
    SEL = acc_ref.shape[0]
    B = o1_ref.shape[0]

    @pl.when(i == 0)
    def _():
        acc_ref[...] = jnp.zeros_like(acc_ref)

    blk = p0_ref[0] + p1_ref[0] + xt_ref[...]
    col = lax.broadcasted_iota(jnp.int32, (SEL, 128), 1) + i * 128
    oh = (sel_ref[...] == col).astype(F32)
    acc_ref[...] += lax.dot_general(oh, blk, (((1,), (0,)), ((), ())),
                                    precision=_PREC,
                                    preferred_element_type=F32)

    @pl.when(i == pl.num_programs(0) - 1)
    def _():
        agg = acc_ref[...]
        h = _proj(_expmap0(agg))
        h = _proj(_expmap0(jax.nn.relu(_logmap0(h))))
        res = _proj(_expmap0(_logmap0(h)))
        o1_ref[...] = res[:B]
        o2_ref[...] = res[B:2 * B]


def _run_stage_f(parts, xt2, sel_b, B):
    NP, d = xt2.shape
    SEL = sel_b.shape[0]
    grid = (NP // 128,)
    return pl.pallas_call(
        _stage_f_body,
        grid=grid,
        in_specs=[
            pl.BlockSpec((1, 128, d), lambda i: (0, i, 0)),
            pl.BlockSpec((1, 128, d), lambda i: (1, i, 0)),
            pl.BlockSpec((128, d), lambda i: (i, 0)),
            pl.BlockSpec((SEL, d), lambda i: (0, 0)),
        ],
        out_specs=[
            pl.BlockSpec((B, d), lambda i: (0, 0)),
            pl.BlockSpec((B, d), lambda i: (0, 0)),
        ],
        out_shape=[
            jax.ShapeDtypeStruct((B, d), F32),
            jax.ShapeDtypeStruct((B, d), F32),
        ],
        scratch_shapes=[pltpu.VMEM((SEL, d), F32)],
    )(parts, parts, xt2, sel_b)


_CH = 128


def _sc_remap(srcp, dstp, qp):
    EP = srcp.shape[0]
    CH = _CH
    NCH = EP // CH
    info = plsc.get_sparse_core_info()
    NC, NS = info.num_cores, info.num_subcores
    NW = NC * NS
    CPW = NCH // NW
    mesh = plsc.VectorSubcoreMesh(core_axis_name="c", subcore_axis_name="s")

    @functools.partial(
        pl.kernel, mesh=mesh,
        out_type=[jax.ShapeDtypeStruct((EP,), jnp.int32),
                  jax.ShapeDtypeStruct((EP,), jnp.int32)],
        scratch_types=[
            pltpu.VMEM((2, CH), jnp.int32),
            pltpu.VMEM((2, CH), jnp.int32),
            pltpu.VMEM((2, CH), jnp.int32),
            pltpu.VMEM((2, CH), jnp.int32),
            pltpu.SemaphoreType.DMA,
            pltpu.SemaphoreType.DMA,
            pltpu.SemaphoreType.DMA,
            pltpu.SemaphoreType.DMA,
            pltpu.SemaphoreType.DMA,
        ],
    )
    def remap_kernel(src_hbm, dst_hbm, qp_hbm, src2_hbm, dst2_hbm,
                     sidx, didx, sidx2, didx2, fs0, fs1, rsem, ws0, ws1):
        fsems = (fs0, fs1)
        wsems = (ws0, ws1)
        c = lax.axis_index("c")
        s = lax.axis_index("s")
        wid = s * NC + c
        base_chunk = wid * CPW

        def fetch_cp(k, p):
            eb = (base_chunk + k) * CH
            return (pltpu.make_async_copy(src_hbm.at[pl.ds(eb, CH)],
                                          sidx.at[p], fsems[p]),
                    pltpu.make_async_copy(dst_hbm.at[pl.ds(eb, CH)],
                                          didx.at[p], fsems[p]))

        def remap_cp(k, p):
            return (pltpu.make_async_copy(qp_hbm.at[sidx.at[p]],
                                          sidx2.at[p], rsem),
                    pltpu.make_async_copy(qp_hbm.at[didx.at[p]],
                                          didx2.at[p], rsem))

        def wb_cp(k, p):
            eb = (base_chunk + k) * CH
            return (pltpu.make_async_copy(sidx2.at[p],
                                          src2_hbm.at[pl.ds(eb, CH)],
                                          wsems[p]),
                    pltpu.make_async_copy(didx2.at[p],
                                          dst2_hbm.at[pl.ds(eb, CH)],
                                          wsems[p]))

        def start2(cp):
            cp[0].start()
            cp[1].start()

        def wait2(cp):
            cp[0].wait()
            cp[1].wait()

        start2(fetch_cp(0, 0))

        def step(i, carry):
            for b in range(2):
                k = 2 * i + b
                nb = 1 - b

                @pl.when(k > 1)
                def _():
                    wait2(wb_cp(k - 2, b))

                @pl.when(k + 1 < CPW)
                def _():
                    start2(fetch_cp(k + 1, nb))

                wait2(fetch_cp(k, b))
                start2(remap_cp(k, b))
                wait2(remap_cp(k, b))
                start2(wb_cp(k, b))
            return carry

        lax.fori_loop(0, CPW // 2, step, 0)
        wait2(wb_cp(CPW - 2, (CPW - 2) % 2))
        wait2(wb_cp(CPW - 1, (CPW - 1) % 2))

    return remap_kernel(srcp, dstp, qp)


def _sc_aggregate(xt, src2p, dst2p, zrows):
    NP, d = xt.shape
    EP = src2p.shape[0]
    CH = _CH
    NCH = EP // CH
    info = plsc.get_sparse_core_info()
    NC, NS = info.num_cores, info.num_subcores
    NW = NC * NS
    CPW = NCH // NW
    ZR = zrows.shape[0]
    RPS = NP // NS
    mesh = plsc.VectorSubcoreMesh(core_axis_name="c", subcore_axis_name="s")

    @functools.partial(
        pl.kernel, mesh=mesh,
        out_type=jax.ShapeDtypeStruct((NC, NP, d), F32),
        scratch_types=[
            pltpu.VMEM((3, CH), jnp.int32),
            pltpu.VMEM((3, CH), jnp.int32),
            pltpu.VMEM((3, CH, d), F32),
            pltpu.VMEM_SHARED((NP, d), F32),
            pltpu.SemaphoreType.DMA,
            pltpu.SemaphoreType.DMA,
            pltpu.SemaphoreType.DMA,
            pltpu.SemaphoreType.DMA,
            pltpu.SemaphoreType.DMA,
            pltpu.SemaphoreType.DMA,
        ],
    )
    def agg_kernel(xt_hbm, src_hbm, dst_hbm, z_hbm, out_hbm,
                   sidx, didx, rows, acc,
                   isem, f0, f1, g0, g1, ssem):
        fsems = (f0, f1)
        gsems = (g0, g1)
        c = lax.axis_index("c")
        s = lax.axis_index("s")
        wid = s * NC + c
        base_chunk = wid * CPW
        row0 = s * RPS
        nfull = RPS // ZR
        rem = RPS - nfull * ZR

        for k in range(nfull):
            pltpu.async_copy(z_hbm, acc.at[pl.ds(row0 + k * ZR, ZR)], isem)
        if rem:
            pltpu.async_copy(z_hbm.at[pl.ds(0, rem)],
                             acc.at[pl.ds(row0 + nfull * ZR, rem)], isem)
        for k in range(nfull):
            pltpu.make_async_copy(z_hbm, acc.at[pl.ds(row0 + k * ZR, ZR)],
                                  isem).wait()
        if rem:
            pltpu.make_async_copy(z_hbm.at[pl.ds(0, rem)],
                                  acc.at[pl.ds(row0 + nfull * ZR, rem)],
                                  isem).wait()
        plsc.subcore_barrier()

        def fetch_cp(k, p):
            eb = (base_chunk + k) * CH
            sl = k % 3
            return (pltpu.make_async_copy(src_hbm.at[pl.ds(eb, CH)],
                                          sidx.at[sl], fsems[p]),
                    pltpu.make_async_copy(dst_hbm.at[pl.ds(eb, CH)],
                                          didx.at[sl], fsems[p]))

        def gather_cp(k, p):
            return pltpu.make_async_copy(xt_hbm.at[sidx.at[k % 3]],
                                         rows.at[k % 3], gsems[p])

        def scatter_cp(k):
            return pltpu.make_async_copy(rows.at[k % 3],
                                         acc.at[didx.at[k % 3]], ssem)

        def start2(cp):
            cp[0].start()
            cp[1].start()

        def wait2(cp):
            cp[0].wait()
            cp[1].wait()

        start2(fetch_cp(0, 0))
        start2(fetch_cp(1, 1))
        wait2(fetch_cp(0, 0))
        gather_cp(0, 0).start()
        wait2(fetch_cp(1, 1))
        gather_cp(1, 1).start()

        def step(i, carry):
            for b in range(2):
                k = 2 * i + b
                nb = 1 - b

                @pl.when(k > 0)
                def _():
                    scatter_cp(k - 1).wait()

                @pl.when(k + 2 < CPW)
                def _():
                    start2(fetch_cp(k + 2, b))

                gather_cp(k, b).wait()
                scatter_cp(k).start(add=True)

                @pl.when(k + 2 < CPW)
                def _():
                    wait2(fetch_cp(k + 2, b))
                    gather_cp(k + 2, b).start()
            return carry

        lax.fori_loop(0, CPW // 2, step, 0)
        scatter_cp(CPW - 1).wait()

        plsc.subcore_barrier()
        pltpu.sync_copy(acc.at[pl.ds(row0, RPS)],
                        out_hbm.at[c, pl.ds(row0, RPS)])

    return agg_kernel(xt, src2p, dst2p, zrows)


def kernel(x_1, x_2, n_1, n_2, edge_index, t1_W1, t1_W2, t2_W1, t2_W2,
           g_W1, g_W2, t1_b1, t1_b2, t2_b1, t2_b2, g_b1, g_b2):
    N1, d = x_1.shape
    N2 = x_2.shape[0]
    N = N1 + N2
    B = n_1.shape[0]
    P1 = -(-N1 // 128) * 128
    P2 = -(-N2 // 128) * 128
    NP = P1 + P2

    n1 = n_1.astype(jnp.int32)
    n2 = n_2.astype(jnp.int32)
    cum = jnp.cumsum(n1 + n2)
    zero = jnp.zeros((1,), jnp.int32)
    C0 = jnp.concatenate([zero, cum[:-1]])
    c1 = jnp.concatenate([zero, jnp.cumsum(n1)[:-1]])
    c2 = jnp.concatenate([zero, jnp.cumsum(n2)[:-1]])
    j = jnp.arange(N, dtype=jnp.int32)
    geB = (j[:, None] >= cum[None, :]).astype(F32)
    ge_ext = jnp.concatenate([jnp.ones((N, 1), F32), geB], axis=1)
    oh = ge_ext[:, :B] - ge_ext[:, 1:B + 1]
    tbl = jnp.stack([C0, n1, c1, c2], axis=1).astype(F32)
    vals = jax.lax.dot_general(oh, tbl, (((1,), (0,)), ((), ())),
                               precision=_PREC)
    C0g = vals[:, 0].astype(jnp.int32)
    n1g = vals[:, 1].astype(jnp.int32)
    c1g = vals[:, 2].astype(jnp.int32)
    c2g = vals[:, 3].astype(jnp.int32)
    within = j - C0g
    perm = jnp.where(within < n1g, c1g + within,
                     N1 + c2g + within - n1g).astype(jnp.int32)
    qp = perm + jnp.where(perm >= N1, P1 - N1, 0).astype(jnp.int32)
    QPAD = -(-(N + 1) // 16) * 16
    qp_pad = jnp.concatenate([qp, jnp.full((QPAD - N,), NP - 1, jnp.int32)])
    a_idx = C0
    b_idx = C0 + n1
    SEL = -(-(2 * B) // 128) * 128
    sel = jnp.concatenate([qp[a_idx], qp[b_idx],
                           jnp.zeros((SEL - 2 * B,), jnp.int32)])
    sel_b = jnp.broadcast_to(sel[:, None], (SEL, d))

    E = edge_index.shape[1]
    CH = _CH
    info = plsc.get_sparse_core_info()
    NW = info.num_cores * info.num_subcores
    GRAN = CH * NW * 2
    EP = -(-E // GRAN) * GRAN
    src = edge_index[0].astype(jnp.int32)
    dst = edge_index[1].astype(jnp.int32)
    if EP != E:
        src = jnp.concatenate([src, jnp.zeros((EP - E,), jnp.int32)])
        dst = jnp.concatenate([dst, jnp.full((EP - E,), N, jnp.int32)])

    zrows = jnp.zeros((CH, d), F32)

    Xs = jnp.stack([jnp.pad(x_1, ((0, P1 - N1), (0, 0))),
                    jnp.pad(x_2, ((0, P2 - N2), (0, 0)))])
    TW1 = jnp.stack([t1_W1, t2_W1])
    TW2 = jnp.stack([t1_W2, t2_W2])
    Tb1 = jnp.stack([t1_b1, t2_b1]).reshape(2, 1, d)
    Tb2 = jnp.stack([t1_b2, t2_b2]).reshape(2, 1, d)
    gb1 = g_b1.reshape(1, d)
    gb2 = g_b2.reshape(1, d)

    src2, dst2 = _sc_remap(src, dst, qp_pad)
    xt1 = _run_stage_a(Xs, TW1, Tb1, TW2, Tb2, g_W1, gb1, NP)
    parts1 = _sc_aggregate(xt1, src2, dst2, zrows)
    xt2 = _run_stage_d(parts1, xt1, g_W2, gb2)
    parts2 = _sc_aggregate(xt2, src2, dst2, zrows)
    o1, o2 = _run_stage_f(parts2, xt2, sel_b.astype(jnp.int32), B)
    return (o1, o2)


# speedup vs baseline: 3.1081x; 1.3070x over previous
import functools

import jax
import jax.numpy as jnp
from jax import lax
from jax.experimental import pallas as pl
from jax.experimental.pallas import tpu as pltpu
from jax.experimental.pallas import tpu_sc as plsc

F32 = jnp.float32
_INV_S = 1.0 / (1.0 + 1e-5) ** 0.5
_MAXN = 1.0 - 1e-5
_PREC = lax.Precision.HIGHEST


def _rnorm(x):
    return jnp.maximum(jnp.sqrt(jnp.sum(x * x, axis=-1, keepdims=True)), 1e-15)


def _artanh(x):
    x = jnp.clip(x, -1.0 + 1e-7, 1.0 - 1e-7)
    return 0.5 * (jnp.log1p(x) - jnp.log1p(-x))


def _proj(x):
    n = _rnorm(x)
    return jnp.where(n > _MAXN, x / n * _MAXN, x)


def _expmap0(u):
    n = _rnorm(u)
    return jnp.tanh(n) * u / n


def _logmap0(p):
    n = _rnorm(p)
    return _artanh(n) * p / n


def _mobius_add(x, y):
    x2 = jnp.sum(x * x, -1, keepdims=True)
    y2 = jnp.sum(y * y, -1, keepdims=True)
    xy = jnp.sum(x * y, -1, keepdims=True)
    num = (1 + 2 * xy + y2) * x + (1 - x2) * y
    den = 1 + 2 * xy + x2 * y2
    return num / jnp.maximum(den, 1e-15)


def _matvecT(x, W):
    return lax.dot_general(x, W, (((1,), (1,)), ((), ())),
                           precision=_PREC, preferred_element_type=F32)


def _mobius_matvec_block(y, W):
    xn = _rnorm(y)
    mx = _matvecT(y, W)
    mn = _rnorm(mx)
    return jnp.tanh(mn / xn * _artanh(xn)) * mx / mn


def _stage_a_body(x_ref, w1_ref, b1_ref, w2_ref, b2_ref, gw_ref, gb_ref,
                  out_ref):
    x = x_ref[0]
    h = jax.nn.relu((_matvecT(x, w1_ref[0]) + b1_ref[0]) * _INV_S)
    h = jax.nn.relu((_matvecT(h, w2_ref[0]) + b2_ref[0]) * _INV_S)
    y = _proj(_expmap0(h))
    mv = _proj(_mobius_matvec_block(y, gw_ref[...]))
    gb = _proj(_expmap0(gb_ref[...]))
    h2 = _proj(_mobius_add(mv, gb))
    out_ref[...] = _logmap0(h2)


def _run_stage_a(Xs, TW1, Tb1, TW2, Tb2, gW, gb, NP):
    T, P, d = Xs.shape
    BR = 624
    PB = P // BR
    grid = (T, PB)
    return pl.pallas_call(
        _stage_a_body,
        grid=grid,
        in_specs=[
            pl.BlockSpec((1, BR, d), lambda t, i: (t, i, 0)),
            pl.BlockSpec((1, d, d), lambda t, i: (t, 0, 0)),
            pl.BlockSpec((1, 1, d), lambda t, i: (t, 0, 0)),
            pl.BlockSpec((1, d, d), lambda t, i: (t, 0, 0)),
            pl.BlockSpec((1, 1, d), lambda t, i: (t, 0, 0)),
            pl.BlockSpec((d, d), lambda t, i: (0, 0)),
            pl.BlockSpec((1, d), lambda t, i: (0, 0)),
        ],
        out_specs=pl.BlockSpec((BR, d), lambda t, i: (t * PB + i, 0)),
        out_shape=jax.ShapeDtypeStruct((NP, d), F32),
    )(Xs, TW1, Tb1, TW2, Tb2, gW, gb)


def _stage_d_body(p0_ref, p1_ref, xt_ref, gw_ref, gb_ref, out_ref):
    agg = p0_ref[0] + p1_ref[0] + xt_ref[...]
    h = _proj(_expmap0(agg))
    h = _proj(_expmap0(jax.nn.relu(_logmap0(h))))
    mv = _proj(_mobius_matvec_block(h, gw_ref[...]))
    gb = _proj(_expmap0(gb_ref[...]))
    out_ref[...] = _logmap0(_proj(_mobius_add(mv, gb)))


def _run_stage_d(parts, xt1, gW, gb):
    NP, d = xt1.shape
    BR = 624
    grid = (NP // BR,)
    return pl.pallas_call(
        _stage_d_body,
        grid=grid,
        in_specs=[
            pl.BlockSpec((1, BR, d), lambda i: (0, i, 0)),
            pl.BlockSpec((1, BR, d), lambda i: (1, i, 0)),
            pl.BlockSpec((BR, d), lambda i: (i, 0)),
            pl.BlockSpec((d, d), lambda i: (0, 0)),
            pl.BlockSpec((1, d), lambda i: (0, 0)),
        ],
        out_specs=pl.BlockSpec((BR, d), lambda i: (i, 0)),
        out_shape=jax.ShapeDtypeStruct((NP, d), F32),
    )(parts, parts, xt1, gW, gb)


def _stage_f_body(p0_ref, p1_ref, xt_ref, sel_ref, o1_ref, o2_ref, acc_ref):
    i = pl.program_id(0)
    SEL = acc_ref.shape[0]
    B = o1_ref.shape[0]

    @pl.when(i == 0)
    def _():
        acc_ref[...] = jnp.zeros_like(acc_ref)

    blk = p0_ref[0] + p1_ref[0] + xt_ref[...]
    BR = blk.shape[0]
    col = lax.broadcasted_iota(jnp.int32, (SEL, BR), 1) + i * BR
    oh = (sel_ref[...] == col).astype(F32)
    acc_ref[...] += lax.dot_general(oh, blk, (((1,), (0,)), ((), ())),
                                    precision=_PREC,
                                    preferred_element_type=F32)

    @pl.when(i == pl.num_programs(0) - 1)
    def _():
        agg = acc_ref[...]
        h = _proj(_expmap0(agg))
        h = _proj(_expmap0(jax.nn.relu(_logmap0(h))))
        res = _proj(_expmap0(_logmap0(h)))
        o1_ref[...] = res[:B]
        o2_ref[...] = res[B:2 * B]


def _run_stage_f(parts, xt2, sel, B):
    NP, d = xt2.shape
    SEL = sel.shape[0]
    BR = 624
    sel_b = jnp.broadcast_to(sel[:, None], (SEL, BR)).astype(jnp.int32)
    grid = (NP // BR,)
    return pl.pallas_call(
        _stage_f_body,
        grid=grid,
        in_specs=[
            pl.BlockSpec((1, BR, d), lambda i: (0, i, 0)),
            pl.BlockSpec((1, BR, d), lambda i: (1, i, 0)),
            pl.BlockSpec((BR, d), lambda i: (i, 0)),
            pl.BlockSpec((SEL, BR), lambda i: (0, 0)),
        ],
        out_specs=[
            pl.BlockSpec((B, d), lambda i: (0, 0)),
            pl.BlockSpec((B, d), lambda i: (0, 0)),
        ],
        out_shape=[
            jax.ShapeDtypeStruct((B, d), F32),
            jax.ShapeDtypeStruct((B, d), F32),
        ],
        scratch_shapes=[pltpu.VMEM((SEL, d), F32)],
    )(parts, parts, xt2, sel_b)


_CH = 128


def _sc_remap(srcp, dstp, qp):
    EP = srcp.shape[0]
    CH = _CH
    NCH = EP // CH
    info = plsc.get_sparse_core_info()
    NC, NS = info.num_cores, info.num_subcores
    NW = NC * NS
    CPW = NCH // NW
    mesh = plsc.VectorSubcoreMesh(core_axis_name="c", subcore_axis_name="s")

    @functools.partial(
        pl.kernel, mesh=mesh,
        out_type=[jax.ShapeDtypeStruct((EP,), jnp.int32),
                  jax.ShapeDtypeStruct((EP,), jnp.int32)],
        scratch_types=[
            pltpu.VMEM((2, CH), jnp.int32),
            pltpu.VMEM((2, CH), jnp.int32),
            pltpu.VMEM((2, CH), jnp.int32),
            pltpu.VMEM((2, CH), jnp.int32),
            pltpu.SemaphoreType.DMA,
            pltpu.SemaphoreType.DMA,
            pltpu.SemaphoreType.DMA,
            pltpu.SemaphoreType.DMA,
            pltpu.SemaphoreType.DMA,
        ],
    )
    def remap_kernel(src_hbm, dst_hbm, qp_hbm, src2_hbm, dst2_hbm,
                     sidx, didx, sidx2, didx2, fs0, fs1, rsem, ws0, ws1):
        fsems = (fs0, fs1)
        wsems = (ws0, ws1)
        c = lax.axis_index("c")
        s = lax.axis_index("s")
        wid = s * NC + c
        base_chunk = wid * CPW

        def fetch_cp(k, p):
            eb = (base_chunk + k) * CH
            return (pltpu.make_async_copy(src_hbm.at[pl.ds(eb, CH)],
                                          sidx.at[p], fsems[p]),
                    pltpu.make_async_copy(dst_hbm.at[pl.ds(eb, CH)],
                                          didx.at[p], fsems[p]))

        def remap_cp(k, p):
            return (pltpu.make_async_copy(qp_hbm.at[sidx.at[p]],
                                          sidx2.at[p], rsem),
                    pltpu.make_async_copy(qp_hbm.at[didx.at[p]],
                                          didx2.at[p], rsem))

        def wb_cp(k, p):
            eb = (base_chunk + k) * CH
            return (pltpu.make_async_copy(sidx2.at[p],
                                          src2_hbm.at[pl.ds(eb, CH)],
                                          wsems[p]),
                    pltpu.make_async_copy(didx2.at[p],
                                          dst2_hbm.at[pl.ds(eb, CH)],
                                          wsems[p]))

        def start2(cp):
            cp[0].start()
            cp[1].start()

        def wait2(cp):
            cp[0].wait()
            cp[1].wait()

        start2(fetch_cp(0, 0))

        def step(i, carry):
            for b in range(2):
                k = 2 * i + b
                nb = 1 - b

                @pl.when(k > 1)
                def _():
                    wait2(wb_cp(k - 2, b))

                @pl.when(k + 1 < CPW)
                def _():
                    start2(fetch_cp(k + 1, nb))

                wait2(fetch_cp(k, b))
                start2(remap_cp(k, b))
                wait2(remap_cp(k, b))
                start2(wb_cp(k, b))
            return carry

        lax.fori_loop(0, CPW // 2, step, 0)
        wait2(wb_cp(CPW - 2, (CPW - 2) % 2))
        wait2(wb_cp(CPW - 1, (CPW - 1) % 2))

    return remap_kernel(srcp, dstp, qp)


def _sc_aggregate(xt, src2p, dst2p):
    NP, d = xt.shape
    EP = src2p.shape[0]
    CH = _CH
    NCH = EP // CH
    info = plsc.get_sparse_core_info()
    NC, NS = info.num_cores, info.num_subcores
    NW = NC * NS
    CPW = NCH // NW
    U = NCH // (3 * NS)
    CPW0 = 2 * U
    CPW1 = U
    ZR = CH
    RPS = NP // NS
    mesh = plsc.VectorSubcoreMesh(core_axis_name="c", subcore_axis_name="s")

    @functools.partial(
        pl.kernel, mesh=mesh,
        out_type=jax.ShapeDtypeStruct((NC, NP, d), F32),
        scratch_types=[
            pltpu.VMEM((3, CH), jnp.int32),
            pltpu.VMEM((3, CH), jnp.int32),
            pltpu.VMEM((3, CH, d), F32),
            pltpu.VMEM_SHARED((NP, d), F32),
            pltpu.SemaphoreType.DMA,
            pltpu.SemaphoreType.DMA,
            pltpu.SemaphoreType.DMA,
            pltpu.SemaphoreType.DMA,
            pltpu.SemaphoreType.DMA,
            pltpu.SemaphoreType.DMA,
        ],
    )
    def agg_kernel(xt_hbm, src_hbm, dst_hbm, out_hbm,
                   sidx, didx, rows, acc,
                   isem, f0, f1, g0, g1, ssem):
        fsems = (f0, f1)
        gsems = (g0, g1)
        c = lax.axis_index("c")
        s = lax.axis_index("s")
        cpw = jnp.where(c == 0, CPW0, CPW1)
        base_chunk = jnp.where(c == 0, s * CPW0, NS * CPW0 + s * CPW1)
        row0 = s * RPS
        nfull = RPS // ZR
        rem = RPS - nfull * ZR

        def zstore(i, carry):
            r = i // (d // 16)
            g = i - r * (d // 16)
            rows[0, r, pl.ds(g * 16, 16)] = jnp.zeros((16,), F32)
            return carry

        lax.fori_loop(0, ZR * (d // 16), zstore, 0)
        for k in range(nfull):
            pltpu.async_copy(rows.at[0], acc.at[pl.ds(row0 + k * ZR, ZR)],
                             isem)
        if rem:
            pltpu.async_copy(rows.at[0, pl.ds(0, rem)],
                             acc.at[pl.ds(row0 + nfull * ZR, rem)], isem)
        for k in range(nfull):
            pltpu.make_async_copy(rows.at[0],
                                  acc.at[pl.ds(row0 + k * ZR, ZR)],
                                  isem).wait()
        if rem:
            pltpu.make_async_copy(rows.at[0, pl.ds(0, rem)],
                                  acc.at[pl.ds(row0 + nfull * ZR, rem)],
                                  isem).wait()
        plsc.subcore_barrier()

        def fetch_cp(k, p):
            eb = (base_chunk + k) * CH
            sl = k % 3
            return (pltpu.make_async_copy(src_hbm.at[pl.ds(eb, CH)],
                                          sidx.at[sl], fsems[p]),
                    pltpu.make_async_copy(dst_hbm.at[pl.ds(eb, CH)],
                                          didx.at[sl], fsems[p]))

        def gather_cp(k, p):
            return pltpu.make_async_copy(xt_hbm.at[sidx.at[k % 3]],
                                         rows.at[k % 3], gsems[p])

        def scatter_cp(k):
            return pltpu.make_async_copy(rows.at[k % 3],
                                         acc.at[didx.at[k % 3]], ssem)

        def start2(cp):
            cp[0].start()
            cp[1].start()

        def wait2(cp):
            cp[0].wait()
            cp[1].wait()

        @pl.when(cpw >= 2)
        def _():
            start2(fetch_cp(0, 0))
            start2(fetch_cp(1, 1))
            wait2(fetch_cp(0, 0))
            gather_cp(0, 0).start()
            wait2(fetch_cp(1, 1))
            gather_cp(1, 1).start()

        def step(i, carry):
            for b in range(2):
                k = 2 * i + b
                nb = 1 - b

                @pl.when(k > 0)
                def _():
                    scatter_cp(k - 1).wait()

                @pl.when(k + 2 < cpw)
                def _():
                    start2(fetch_cp(k + 2, b))

                gather_cp(k, b).wait()
                scatter_cp(k).start(add=True)

                @pl.when(k + 2 < cpw)
                def _():
                    wait2(fetch_cp(k + 2, b))
                    gather_cp(k + 2, b).start()
            return carry

        lax.fori_loop(0, cpw // 2, step, 0)

        @pl.when(cpw > 0)
        def _():
            scatter_cp(cpw - 1).wait()

        plsc.subcore_barrier()
        pltpu.sync_copy(acc.at[pl.ds(row0, RPS)],
                        out_hbm.at[c, pl.ds(row0, RPS)])

    return agg_kernel(xt, src2p, dst2p)


def kernel(x_1, x_2, n_1, n_2, edge_index, t1_W1, t1_W2, t2_W1, t2_W2,
           g_W1, g_W2, t1_b1, t1_b2, t2_b1, t2_b2, g_b1, g_b2):
    N1, d = x_1.shape
    N2 = x_2.shape[0]
    N = N1 + N2
    B = n_1.shape[0]
    P1 = -(-N1 // 128) * 128
    P2 = -(-N2 // 128) * 128
    NP = P1 + P2

    n1 = n_1.astype(jnp.int32)
    n2 = n_2.astype(jnp.int32)
    cum = jnp.cumsum(n1 + n2)
    zero = jnp.zeros((1,), jnp.int32)
    C0 = jnp.concatenate([zero, cum[:-1]])
    c1 = jnp.concatenate([zero, jnp.cumsum(n1)[:-1]])
    c2 = jnp.concatenate([zero, jnp.cumsum(n2)[:-1]])
    j = jnp.arange(N, dtype=jnp.int32)
    geB = (j[:, None] >= cum[None, :]).astype(F32)
    ge_ext = jnp.concatenate([jnp.ones((N, 1), F32), geB], axis=1)
    oh = ge_ext[:, :B] - ge_ext[:, 1:B + 1]
    tbl = jnp.stack([C0, n1, c1, c2], axis=1).astype(F32)
    vals = jax.lax.dot_general(oh, tbl, (((1,), (0,)), ((), ())),
                               precision=_PREC)
    C0g = vals[:, 0].astype(jnp.int32)
    n1g = vals[:, 1].astype(jnp.int32)
    c1g = vals[:, 2].astype(jnp.int32)
    c2g = vals[:, 3].astype(jnp.int32)
    within = j - C0g
    perm = jnp.where(within < n1g, c1g + within,
                     N1 + c2g + within - n1g).astype(jnp.int32)
    qp = perm + jnp.where(perm >= N1, P1 - N1, 0).astype(jnp.int32)
    QPAD = -(-(N + 1) // 16) * 16
    qp_pad = jnp.concatenate([qp, jnp.full((QPAD - N,), NP - 1, jnp.int32)])
    a_idx = C0
    b_idx = C0 + n1
    SEL = -(-(2 * B) // 128) * 128
    sel = jnp.concatenate([qp[a_idx], qp[b_idx],
                           jnp.zeros((SEL - 2 * B,), jnp.int32)])

    E = edge_index.shape[1]
    CH = _CH
    info = plsc.get_sparse_core_info()
    NW = info.num_cores * info.num_subcores
    GRAN = CH * NW * 3
    EP = -(-E // GRAN) * GRAN
    src = edge_index[0].astype(jnp.int32)
    dst = edge_index[1].astype(jnp.int32)
    if EP != E:
        src = jnp.concatenate([src, jnp.zeros((EP - E,), jnp.int32)])
        dst = jnp.concatenate([dst, jnp.full((EP - E,), N, jnp.int32)])

    Xs = jnp.stack([jnp.pad(x_1, ((0, P1 - N1), (0, 0))),
                    jnp.pad(x_2, ((0, P2 - N2), (0, 0)))])
    TW1 = jnp.stack([t1_W1, t2_W1])
    TW2 = jnp.stack([t1_W2, t2_W2])
    Tb1 = jnp.stack([t1_b1, t2_b1]).reshape(2, 1, d)
    Tb2 = jnp.stack([t1_b2, t2_b2]).reshape(2, 1, d)
    gb1 = g_b1.reshape(1, d)
    gb2 = g_b2.reshape(1, d)

    src2, dst2 = _sc_remap(src, dst, qp_pad)
    xt1 = _run_stage_a(Xs, TW1, Tb1, TW2, Tb2, g_W1, gb1, NP)
    parts1 = _sc_aggregate(xt1, src2, dst2)
    xt2 = _run_stage_d(parts1, xt1, g_W2, gb2)
    parts2 = _sc_aggregate(xt2, src2, dst2)
    o1, o2 = _run_stage_f(parts2, xt2, sel, B)
    return (o1, o2)
